# Initial kernel scaffold; baseline (speedup 1.0000x reference)
import jax
import jax.numpy as jnp
from jax.experimental import pallas as pl


def kernel(x, edge_index, W_l, b_l, W_r, Ws, Wt):
    raise NotImplementedError("write your pallas kernel here")


# speedup vs baseline: 4.4356x; 4.4356x over previous
import jax
import jax.numpy as jnp
from jax import lax
from jax.experimental import pallas as pl
from jax.experimental.pallas import tpu as pltpu
from jax.experimental.pallas import tpu_sc as plsc

N = 10000
E = 320000
D = 128
NC = 2
NS = 16
NW = NC * NS
EPW = E // NW
K = 80
NCH = EPW // K
NCHT = N // K


def _relu_body(x_ref, h_ref):
    h_ref[...] = jnp.maximum(x_ref[...], 0.0)


def _relu(x):
    bn = 1000
    return pl.pallas_call(
        _relu_body,
        grid=(N // bn,),
        in_specs=[pl.BlockSpec((bn, D), lambda i: (i, 0))],
        out_specs=pl.BlockSpec((bn, D), lambda i: (i, 0)),
        out_shape=jax.ShapeDtypeStruct((N, D), jnp.float32),
    )(x)


def _sc_agg_body(h_hbm, src_hbm, dst_hbm, zeros_hbm, agg_out,
                 acc_sh, s_buf, d_buf, buf0, semg):
    c = lax.axis_index("c")
    s = lax.axis_index("s")
    ebase = (c * NS + s) * EPW

    pltpu.sync_copy(zeros_hbm, buf0)

    for b in range(7):
        pltpu.sync_copy(buf0, acc_sh.at[pl.ds((s + NS * b) * K, K)])
    t = 112 + lax.rem(s, 13)
    pltpu.sync_copy(buf0, acc_sh.at[pl.ds(t * K, K)])

    plsc.subcore_barrier()

    def chunk(ci, carry):
        e0 = ebase + ci * K
        pltpu.sync_copy(src_hbm.at[pl.ds(e0, K)], s_buf)
        pltpu.sync_copy(dst_hbm.at[pl.ds(e0, K)], d_buf)
        pltpu.make_async_copy(h_hbm.at[s_buf], buf0, semg).start()
        pltpu.make_async_copy(h_hbm.at[s_buf], buf0, semg).wait()
        pltpu.sync_copy(buf0, acc_sh.at[d_buf], add=True)
        return carry
    lax.fori_loop(0, NCH, chunk, 0)

    plsc.subcore_barrier()

    for b in range(7):
        ch = s + NS * b
        pltpu.sync_copy(acc_sh.at[pl.ds(ch * K, K)], buf0)
        pltpu.sync_copy(buf0, agg_out.at[c, pl.ds(ch * K, K)])
    t = 112 + lax.rem(s, 13)
    pltpu.sync_copy(acc_sh.at[pl.ds(t * K, K)], buf0)
    pltpu.sync_copy(buf0, agg_out.at[c, pl.ds(t * K, K)])


def _sc_aggregate(h, srcf, dstf, zeroskd):
    mesh = plsc.VectorSubcoreMesh(core_axis_name="c", subcore_axis_name="s")
    f = pl.kernel(
        _sc_agg_body,
        out_type=jax.ShapeDtypeStruct((NC, N, D), jnp.float32),
        mesh=mesh,
        scratch_types=[
            pltpu.VMEM_SHARED((N, D), jnp.float32),
            pltpu.VMEM((K,), jnp.int32),
            pltpu.VMEM((K,), jnp.int32),
            pltpu.VMEM((K, D), jnp.float32),
            pltpu.SemaphoreType.DMA,
        ],
    )
    return f(h, srcf, dstf, zeroskd)


def _sc_cnt_body(dst_hbm, ones_hbm, zeros_hbm, cnt_out,
                 cnt_sh, d_buf, ones_b, cb):
    c = lax.axis_index("c")
    s = lax.axis_index("s")
    ebase = (c * NS + s) * EPW

    pltpu.sync_copy(zeros_hbm, cb)

    for b in range(7):
        pltpu.sync_copy(cb, cnt_sh.at[pl.ds((s + NS * b) * K, K)])
    t = 112 + lax.rem(s, 13)
    pltpu.sync_copy(cb, cnt_sh.at[pl.ds(t * K, K)])

    pltpu.sync_copy(ones_hbm, ones_b)

    plsc.subcore_barrier()

    def chunk(ci, carry):
        e0 = ebase + ci * K
        pltpu.sync_copy(dst_hbm.at[pl.ds(e0, K)], d_buf)
        pltpu.sync_copy(ones_b, cnt_sh.at[d_buf], add=True)
        return carry
    lax.fori_loop(0, NCH, chunk, 0)

    plsc.subcore_barrier()

    for b in range(7):
        ch = s + NS * b
        pltpu.sync_copy(cnt_sh.at[pl.ds(ch * K, K)], cb)
        pltpu.sync_copy(cb, cnt_out.at[c, pl.ds(ch * K, K)])
    t = 112 + lax.rem(s, 13)
    pltpu.sync_copy(cnt_sh.at[pl.ds(t * K, K)], cb)
    pltpu.sync_copy(cb, cnt_out.at[c, pl.ds(t * K, K)])


def _sc_counts(dstf, ones128, zeroskd):
    mesh = plsc.VectorSubcoreMesh(core_axis_name="c", subcore_axis_name="s")
    f = pl.kernel(
        _sc_cnt_body,
        out_type=jax.ShapeDtypeStruct((NC, N, D), jnp.float32),
        mesh=mesh,
        scratch_types=[
            pltpu.VMEM_SHARED((N, D), jnp.float32),
            pltpu.VMEM((K,), jnp.int32),
            pltpu.VMEM((K, D), jnp.float32),
            pltpu.VMEM((K, D), jnp.float32),
        ],
    )
    return f(dstf, ones128, zeroskd)


def _head_body(h_ref, agg_ref, cnt_ref, wl_ref, bl_ref, wr_ref, ws_ref,
               wt_ref, o1_ref, os_ref, ot_ref):
    h = h_ref[...]
    agg = agg_ref[0] + agg_ref[1]
    cnt = cnt_ref[0, :, 0:1] + cnt_ref[1, :, 0:1]
    mean = agg / jnp.maximum(cnt, 1.0)
    o1 = (jnp.dot(mean, wl_ref[...], preferred_element_type=jnp.float32)
          + bl_ref[...]
          + jnp.dot(h, wr_ref[...], preferred_element_type=jnp.float32))
    o1_ref[...] = o1
    zn = o1 / jnp.maximum(
        jnp.sqrt(jnp.sum(o1 * o1, axis=1, keepdims=True)), 1e-12)
    ws = ws_ref[...]
    wsn = ws / jnp.maximum(
        jnp.sqrt(jnp.sum(ws * ws, axis=0, keepdims=True)), 1e-12)
    wt = wt_ref[...]
    wtn = wt / jnp.maximum(
        jnp.sqrt(jnp.sum(wt * wt, axis=0, keepdims=True)), 1e-12)
    os_ref[...] = jnp.dot(zn, wsn, preferred_element_type=jnp.float32)
    ot_ref[...] = jnp.dot(zn, wtn, preferred_element_type=jnp.float32)


def _head(h, agg_parts, cnt_parts, W_l, b_l, W_r, Ws, Wt):
    bn = 400
    s_cls = Ws.shape[1]
    t_cls = Wt.shape[1]
    return pl.pallas_call(
        _head_body,
        grid=(N // bn,),
        in_specs=[
            pl.BlockSpec((bn, D), lambda i: (i, 0)),
            pl.BlockSpec((NC, bn, D), lambda i: (0, i, 0)),
            pl.BlockSpec((NC, bn, D), lambda i: (0, i, 0)),
            pl.BlockSpec((D, D), lambda i: (0, 0)),
            pl.BlockSpec((1, D), lambda i: (0, 0)),
            pl.BlockSpec((D, D), lambda i: (0, 0)),
            pl.BlockSpec((D, s_cls), lambda i: (0, 0)),
            pl.BlockSpec((D, t_cls), lambda i: (0, 0)),
        ],
        out_specs=[
            pl.BlockSpec((bn, D), lambda i: (i, 0)),
            pl.BlockSpec((bn, s_cls), lambda i: (i, 0)),
            pl.BlockSpec((bn, t_cls), lambda i: (i, 0)),
        ],
        out_shape=[
            jax.ShapeDtypeStruct((N, D), jnp.float32),
            jax.ShapeDtypeStruct((N, s_cls), jnp.float32),
            jax.ShapeDtypeStruct((N, t_cls), jnp.float32),
        ],
    )(h, agg_parts, cnt_parts, W_l, b_l, W_r, Ws, Wt)


def kernel(x, edge_index, W_l, b_l, W_r, Ws, Wt):
    h = _relu(x)
    zeroskd = jnp.zeros((K, D), jnp.float32)
    ones128 = jnp.zeros((K, D), jnp.float32).at[:, 0].set(1.0)
    agg_parts = _sc_aggregate(h, edge_index[0], edge_index[1], zeroskd)
    cnt_parts = _sc_counts(edge_index[1], ones128, zeroskd)
    out1, out_s, out_t = _head(h, agg_parts, cnt_parts, W_l,
                               b_l.reshape(1, D), W_r, Ws, Wt)
    return (out1, out_s, out_t)
